# Initial kernel scaffold; baseline (speedup 1.0000x reference)
import jax
import jax.numpy as jnp
from jax.experimental import pallas as pl


def kernel(H, src, dst, W, gamma, beta):
    raise NotImplementedError("write your pallas kernel here")


# speedup vs baseline: 4.6964x; 4.6964x over previous
import functools

import jax
import jax.numpy as jnp
from jax import lax
from jax.experimental import pallas as pl
from jax.experimental.pallas import tpu as pltpu
from jax.experimental.pallas import tpu_sc as plsc

D = 128
N_NODES = 10000
NC, NS = 2, 16
NW = NC * NS
ROWS_PER_TILE = 632
N_PAD = NS * ROWS_PER_TILE
CHUNK = 128
N_EDGES = 320000
CPT = -(-N_EDGES // (NW * CHUNK))
EDGES_PAD = NW * CPT * CHUNK

ROW_BLK = 1000


def _mm_body(h_ref, w_ref, o_ref):
    o_ref[...] = lax.dot_general(
        h_ref[...], w_ref[...], (((1,), (1,)), ((), ())),
        preferred_element_type=jnp.float32)


def _fin_body(h_ref, a0_ref, a1_ref, g_ref, b_ref, o_ref):
    agg = a0_ref[...] + a1_ref[...]
    ge = 0.5 * agg * (1.0 + lax.erf(agg * 0.7071067811865476))
    x = h_ref[...] + ge
    mu = jnp.mean(x, axis=1, keepdims=True)
    xc = x - mu
    var = jnp.mean(xc * xc, axis=1, keepdims=True)
    y = xc * lax.rsqrt(var + 1e-5)
    o_ref[...] = y * g_ref[...] + b_ref[...]


def _sc_body(m_hbm, src_hbm, dst_hbm, zero_hbm, out_hbm,
             src_v, dst_v, rows_v, sem, shared):
    cid = lax.axis_index("c")
    sid = lax.axis_index("s")
    wid = sid * NC + cid
    row0 = sid * ROWS_PER_TILE

    pltpu.sync_copy(zero_hbm.at[pl.ds(row0, ROWS_PER_TILE)],
                    shared.at[pl.ds(row0, ROWS_PER_TILE)])
    pltpu.sync_copy(src_hbm.at[wid], src_v)
    pltpu.sync_copy(dst_hbm.at[wid], dst_v)
    plsc.subcore_barrier()

    def chunk_body(g, carry):
        pltpu.async_copy(m_hbm.at[src_v.at[g]], rows_v, sem).wait()
        pltpu.sync_copy(rows_v, shared.at[dst_v.at[g]], add=True)
        return carry

    lax.fori_loop(0, CPT, chunk_body, 0)
    plsc.subcore_barrier()
    pltpu.sync_copy(shared.at[pl.ds(row0, ROWS_PER_TILE)],
                    out_hbm.at[cid, pl.ds(row0, ROWS_PER_TILE)])


_sc_scatter = pl.kernel(
    _sc_body,
    out_type=jax.ShapeDtypeStruct((NC, N_PAD, D), jnp.float32),
    mesh=plsc.VectorSubcoreMesh(core_axis_name="c", subcore_axis_name="s"),
    scratch_types=[
        pltpu.VMEM((CPT, CHUNK), jnp.int32),
        pltpu.VMEM((CPT, CHUNK), jnp.int32),
        pltpu.VMEM((CHUNK, D), jnp.float32),
        pltpu.SemaphoreType.DMA,
        pltpu.VMEM_SHARED((N_PAD, D), jnp.float32),
    ],
)


def kernel(H, src, dst, W, gamma, beta):
    H2 = H.reshape(N_NODES, D)

    m = pl.pallas_call(
        _mm_body,
        out_shape=jax.ShapeDtypeStruct((N_NODES, D), jnp.float32),
        grid=(N_NODES // ROW_BLK,),
        in_specs=[pl.BlockSpec((ROW_BLK, D), lambda i: (i, 0)),
                  pl.BlockSpec((D, D), lambda i: (0, 0))],
        out_specs=pl.BlockSpec((ROW_BLK, D), lambda i: (i, 0)),
    )(H2, W)

    pad = EDGES_PAD - src.shape[0]
    src3 = jnp.concatenate(
        [src.astype(jnp.int32), jnp.zeros((pad,), jnp.int32)]
    ).reshape(NW, CPT, CHUNK)
    dst3 = jnp.concatenate(
        [dst.astype(jnp.int32), jnp.full((pad,), N_NODES, jnp.int32)]
    ).reshape(NW, CPT, CHUNK)
    zeros = jnp.zeros((N_PAD, D), jnp.float32)

    parts = _sc_scatter(m, src3, dst3, zeros)

    out = pl.pallas_call(
        _fin_body,
        out_shape=jax.ShapeDtypeStruct((N_NODES, D), jnp.float32),
        grid=(N_NODES // ROW_BLK,),
        in_specs=[pl.BlockSpec((ROW_BLK, D), lambda i: (i, 0)),
                  pl.BlockSpec((ROW_BLK, D), lambda i: (i, 0)),
                  pl.BlockSpec((ROW_BLK, D), lambda i: (i, 0)),
                  pl.BlockSpec((1, D), lambda i: (0, 0)),
                  pl.BlockSpec((1, D), lambda i: (0, 0))],
        out_specs=pl.BlockSpec((ROW_BLK, D), lambda i: (i, 0)),
    )(H2, parts[0, :N_NODES], parts[1, :N_NODES],
      gamma.reshape(1, D), beta.reshape(1, D))

    return out.reshape(1, N_NODES, D)
